# baseline (speedup 1.0000x reference)
import jax
import jax.numpy as jnp
from jax.experimental import pallas as pl
from jax.experimental.pallas import tpu as pltpu

B, N, D = 32, 1024, 768
S = 8
H = 1536
ITERS = 3
EPS = 1e-8
G = 4
NG = B // G
P = 2
NH = N // P
GS = G * S
BS = B * S
SCALE = D ** -0.5


def _ln(x, g, b):
    m = jnp.mean(x, axis=-1, keepdims=True)
    v = jnp.mean((x - m) ** 2, axis=-1, keepdims=True)
    return (x - m) * jax.lax.rsqrt(v + 1e-5) * g + b


def _norm_rows(x):
    m = jnp.mean(x, axis=-1, keepdims=True)
    v = jnp.mean((x - m) ** 2, axis=-1, keepdims=True)
    return (x - m) * jax.lax.rsqrt(v + 1e-5)


def _soft_part(q_b, k_h, v_h):
    dots = jax.lax.dot_general(
        q_b, k_h, (((1,), (1,)), ((), ())),
        preferred_element_type=jnp.float32) * SCALE
    e = jnp.exp(dots - jnp.max(dots, axis=0, keepdims=True))
    p = e / jnp.sum(e, axis=0, keepdims=True)
    u = jnp.dot(p.astype(jnp.bfloat16), v_h,
                preferred_element_type=jnp.float32)
    return u, jnp.sum(p, axis=1, keepdims=True)


def _attend(q_b, k_ref, v_ref, gi_):
    us = [_soft_part(q_b, k_ref[p_, gi_], v_ref[p_, gi_]) for p_ in range(P)]
    u = sum(x[0] for x in us)
    s = sum(x[1] for x in us)
    return u / (s + N * EPS)


def _proj_kernel(x_ref, noise_ref, mu_ref, sigma_ref,
                 WkT_ref, bk_ref, WvT_ref, bv_ref, WqT_ref, bq_ref,
                 g_s_ref, b_s_ref,
                 k_ref, v_ref, upd0_ref):
    bf = jnp.bfloat16
    h = pl.program_id(1)
    xh = _norm_rows(x_ref[...].reshape(G * NH, D)).astype(bf)
    kh = (jnp.dot(xh, WkT_ref[...], preferred_element_type=jnp.float32)
          + bk_ref[...]).astype(bf)
    vh = (jnp.dot(xh, WvT_ref[...], preferred_element_type=jnp.float32)
          + bv_ref[...]).astype(bf)
    k_ref[pl.ds(h, 1)] = kh.reshape(1, G, NH, D)
    v_ref[pl.ds(h, 1)] = vh.reshape(1, G, NH, D)

    @pl.when(h == P - 1)
    def _attn0():
        slots0 = mu_ref[0] + sigma_ref[0] * noise_ref[...].reshape(GS, D)
        q0 = (jnp.dot(_ln(slots0, g_s_ref[...], b_s_ref[...]).astype(bf),
                      WqT_ref[...], preferred_element_type=jnp.float32)
              + bq_ref[...]).astype(bf)
        for gi_ in range(G):
            upd0_ref[gi_] = _attend(q0[gi_ * S:(gi_ + 1) * S, :],
                                    k_ref, v_ref, gi_)


def _gru_ff(upd, slots_prev, WihT_ref, WhhT_ref, bih_ref, bhh_ref,
            W1T_ref, b1_ref, W2T_ref, b2_ref, g_ff_ref, b_ff_ref):
    bf = jnp.bfloat16
    gi = jnp.dot(upd.astype(bf), WihT_ref[...],
                 preferred_element_type=jnp.float32) + bih_ref[...]
    gh = jnp.dot(slots_prev.astype(bf), WhhT_ref[...],
                 preferred_element_type=jnp.float32) + bhh_ref[...]
    r = jax.nn.sigmoid(gi[:, :D] + gh[:, :D])
    z = jax.nn.sigmoid(gi[:, D:2 * D] + gh[:, D:2 * D])
    n_ = jnp.tanh(gi[:, 2 * D:] + r * gh[:, 2 * D:])
    slots = (1.0 - z) * n_ + z * slots_prev
    ffx = _ln(slots, g_ff_ref[...], b_ff_ref[...]).astype(bf)
    h1 = jax.nn.relu(jnp.dot(ffx, W1T_ref[...],
                             preferred_element_type=jnp.float32)
                     + b1_ref[...]).astype(bf)
    return slots + jnp.dot(h1, W2T_ref[...],
                           preferred_element_type=jnp.float32) + b2_ref[...]


def _iter_kernel(k_ref, v_ref, upd0_ref, noise_ref, mu_ref, sigma_ref,
                 WqT_ref, bq_ref, WihT_ref, WhhT_ref, bih_ref, bhh_ref,
                 W1T_ref, b1_ref, W2T_ref, b2_ref,
                 g_s_ref, b_s_ref, g_ff_ref, b_ff_ref,
                 out_ref, slots_sc, upd_sc, q_sc):
    bf = jnp.bfloat16
    j = pl.program_id(0)
    g = pl.program_id(1)
    gru_args = (WihT_ref, WhhT_ref, bih_ref, bhh_ref,
                W1T_ref, b1_ref, W2T_ref, b2_ref, g_ff_ref, b_ff_ref)

    def _q_of(slots):
        return (jnp.dot(_ln(slots, g_s_ref[...], b_s_ref[...]).astype(bf),
                        WqT_ref[...], preferred_element_type=jnp.float32)
                + bq_ref[...])

    @pl.when(jnp.logical_and(j == 0, g == 0))
    def _init():
        slots0 = mu_ref[0] + sigma_ref[0] * noise_ref[...].reshape(BS, D)
        slots = _gru_ff(upd0_ref[...].reshape(BS, D), slots0, *gru_args)
        slots_sc[...] = slots
        q_sc[...] = _q_of(slots)

    for gi_ in range(G):
        upd_sc[pl.ds(g * GS + gi_ * S, S), :] = _attend(
            q_sc[pl.ds(g * GS + gi_ * S, S), :].astype(bf),
            k_ref, v_ref, gi_)

    @pl.when(g == NG - 1)
    def _global():
        slots = _gru_ff(upd_sc[...], slots_sc[...], *gru_args)

        @pl.when(j < 1)
        def _next():
            slots_sc[...] = slots
            q_sc[...] = _q_of(slots)

        @pl.when(j == 1)
        def _emit():
            out_ref[...] = slots.reshape(B, S, D)


@jax.jit
def kernel(x, slots_noise, mu, logsigma, Wq, bq, Wk, bk, Wv, bv,
           W_ih, W_hh, b_ih, b_hh, W1, b1, W2, b2,
           g_in, b_in, g_slots, b_slots, g_ff, b_ff):
    bf = jnp.bfloat16
    row = lambda a: a.reshape(1, -1)
    full = lambda s, n: pl.BlockSpec(s, lambda *_: (0,) * n)
    sigma = jnp.exp(logsigma)
    WqT = Wq.T.astype(bf)
    WkT_eff = (g_in[:, None] * Wk.T).astype(bf)
    WvT_eff = (g_in[:, None] * Wv.T).astype(bf)
    bk_eff = row(bk + b_in @ Wk.T)
    bv_eff = row(bv + b_in @ Wv.T)

    k, v, upd0 = pl.pallas_call(
        _proj_kernel,
        grid=(NG, P),
        in_specs=[
            pl.BlockSpec((G, NH, D), lambda g, h: (g, h, 0)),
            pl.BlockSpec((G, S, D), lambda g, h: (g, 0, 0)),
            full((1, 1, D), 3), full((1, 1, D), 3),
            full((D, D), 2), full((1, D), 2),
            full((D, D), 2), full((1, D), 2),
            full((D, D), 2), full((1, D), 2),
            full((1, D), 2), full((1, D), 2),
        ],
        out_specs=[
            pl.BlockSpec((P, G, NH, D), lambda g, h: (0, g, 0, 0)),
            pl.BlockSpec((P, G, NH, D), lambda g, h: (0, g, 0, 0)),
            pl.BlockSpec((G, S, D), lambda g, h: (g, 0, 0)),
        ],
        out_shape=[
            jax.ShapeDtypeStruct((P, B, NH, D), bf),
            jax.ShapeDtypeStruct((P, B, NH, D), bf),
            jax.ShapeDtypeStruct((B, S, D), jnp.float32),
        ],
    )(x, slots_noise, mu, sigma,
      WkT_eff, bk_eff, WvT_eff, bv_eff, WqT, row(bq),
      row(g_slots), row(b_slots))

    out = pl.pallas_call(
        _iter_kernel,
        grid=(ITERS - 1, NG),
        in_specs=[
            pl.BlockSpec((P, G, NH, D), lambda j, g: (0, g, 0, 0)),
            pl.BlockSpec((P, G, NH, D), lambda j, g: (0, g, 0, 0)),
            full((B, S, D), 3),
            full((B, S, D), 3),
            full((1, 1, D), 3), full((1, 1, D), 3),
            full((D, D), 2), full((1, D), 2),
            full((D, 3 * D), 2), full((D, 3 * D), 2),
            full((1, 3 * D), 2), full((1, 3 * D), 2),
            full((D, H), 2), full((1, H), 2),
            full((H, D), 2), full((1, D), 2),
            full((1, D), 2), full((1, D), 2),
            full((1, D), 2), full((1, D), 2),
        ],
        out_specs=full((B, S, D), 3),
        out_shape=jax.ShapeDtypeStruct((B, S, D), jnp.float32),
        scratch_shapes=[
            pltpu.VMEM((BS, D), jnp.float32),
            pltpu.VMEM((BS, D), jnp.float32),
            pltpu.VMEM((BS, D), jnp.float32),
        ],
    )(k, v, upd0, slots_noise, mu, sigma,
      WqT, row(bq),
      W_ih.T.astype(bf), W_hh.T.astype(bf), row(b_ih), row(b_hh),
      W1.T.astype(bf), row(b1), W2.T.astype(bf), row(b2),
      row(g_slots), row(b_slots), row(g_ff), row(b_ff))
    return out


# speedup vs baseline: 1.0227x; 1.0227x over previous
import jax
import jax.numpy as jnp
from jax.experimental import pallas as pl
from jax.experimental.pallas import tpu as pltpu

B, N, D = 32, 1024, 768
S = 8
H = 1536
ITERS = 3
EPS = 1e-8
G = 4
NG = B // G
P = 2
NH = N // P
GS = G * S
BS = B * S
SCALE = D ** -0.5


def _dotT(a, w):
    return jax.lax.dot_general(a, w, (((1,), (1,)), ((), ())),
                               preferred_element_type=jnp.float32)


def _ln(x, g, b):
    m = jnp.mean(x, axis=-1, keepdims=True)
    v = jnp.mean((x - m) ** 2, axis=-1, keepdims=True)
    return (x - m) * jax.lax.rsqrt(v + 1e-5) * g + b


def _norm_rows(x):
    m = jnp.mean(x, axis=-1, keepdims=True)
    v = jnp.mean((x - m) ** 2, axis=-1, keepdims=True)
    return (x - m) * jax.lax.rsqrt(v + 1e-5)


def _soft_part(q_b, k_h, v_h):
    dots = jax.lax.dot_general(
        q_b, k_h, (((1,), (1,)), ((), ())),
        preferred_element_type=jnp.float32) * SCALE
    e = jnp.exp(dots - jnp.max(dots, axis=0, keepdims=True))
    p = e / jnp.sum(e, axis=0, keepdims=True)
    u = jnp.dot(p.astype(jnp.bfloat16), v_h,
                preferred_element_type=jnp.float32)
    return u, jnp.sum(p, axis=1, keepdims=True)


def _attend(q_b, k_ref, v_ref, gi_):
    us = [_soft_part(q_b, k_ref[p_, gi_], v_ref[p_, gi_]) for p_ in range(P)]
    u = sum(x[0] for x in us)
    s = sum(x[1] for x in us)
    return u / (s + N * EPS)


def _proj_kernel(x_ref, noise_ref, mu_ref, sigma_ref,
                 Wk_ref, bk_ref, Wv_ref, bv_ref, Wq_ref, bq_ref,
                 g_s_ref, b_s_ref,
                 k_ref, v_ref, upd0_ref):
    bf = jnp.bfloat16
    h = pl.program_id(1)
    xh = _norm_rows(x_ref[...].reshape(G * NH, D)).astype(bf)
    kh = (_dotT(xh, Wk_ref[...]) + bk_ref[...]).astype(bf)
    vh = (_dotT(xh, Wv_ref[...]) + bv_ref[...]).astype(bf)
    k_ref[pl.ds(h, 1)] = kh.reshape(1, G, NH, D)
    v_ref[pl.ds(h, 1)] = vh.reshape(1, G, NH, D)

    @pl.when(h == P - 1)
    def _attn0():
        slots0 = mu_ref[0] + sigma_ref[0] * noise_ref[...].reshape(GS, D)
        q0 = (_dotT(_ln(slots0, g_s_ref[...], b_s_ref[...]).astype(bf),
                    Wq_ref[...]) + bq_ref[...]).astype(bf)
        for gi_ in range(G):
            upd0_ref[gi_] = _attend(q0[gi_ * S:(gi_ + 1) * S, :],
                                    k_ref, v_ref, gi_)


def _gru_ff(upd, slots_prev, Wih_ref, Whh_ref, bih_ref, bhh_ref,
            W1_ref, b1_ref, W2_ref, b2_ref, g_ff_ref, b_ff_ref):
    bf = jnp.bfloat16
    gi = _dotT(upd.astype(bf), Wih_ref[...]) + bih_ref[...]
    gh = _dotT(slots_prev.astype(bf), Whh_ref[...]) + bhh_ref[...]
    r = jax.nn.sigmoid(gi[:, :D] + gh[:, :D])
    z = jax.nn.sigmoid(gi[:, D:2 * D] + gh[:, D:2 * D])
    n_ = jnp.tanh(gi[:, 2 * D:] + r * gh[:, 2 * D:])
    slots = (1.0 - z) * n_ + z * slots_prev
    ffx = _ln(slots, g_ff_ref[...], b_ff_ref[...]).astype(bf)
    h1 = jax.nn.relu(_dotT(ffx, W1_ref[...]) + b1_ref[...]).astype(bf)
    return slots + _dotT(h1, W2_ref[...]) + b2_ref[...]


def _iter_kernel(k_ref, v_ref, upd0_ref, noise_ref, mu_ref, sigma_ref,
                 Wq_ref, bq_ref, Wih_ref, Whh_ref, bih_ref, bhh_ref,
                 W1_ref, b1_ref, W2_ref, b2_ref,
                 g_s_ref, b_s_ref, g_ff_ref, b_ff_ref,
                 out_ref, slots_sc, upd_sc, q_sc):
    bf = jnp.bfloat16
    j = pl.program_id(0)
    g = pl.program_id(1)
    gru_args = (Wih_ref, Whh_ref, bih_ref, bhh_ref,
                W1_ref, b1_ref, W2_ref, b2_ref, g_ff_ref, b_ff_ref)

    def _q_of(slots):
        return (_dotT(_ln(slots, g_s_ref[...], b_s_ref[...]).astype(bf),
                      Wq_ref[...]) + bq_ref[...])

    @pl.when(jnp.logical_and(j == 0, g == 0))
    def _init():
        slots0 = mu_ref[0] + sigma_ref[0] * noise_ref[...].reshape(BS, D)
        slots = _gru_ff(upd0_ref[...].reshape(BS, D), slots0, *gru_args)
        slots_sc[...] = slots
        q_sc[...] = _q_of(slots)

    for gi_ in range(G):
        upd_sc[pl.ds(g * GS + gi_ * S, S), :] = _attend(
            q_sc[pl.ds(g * GS + gi_ * S, S), :].astype(bf),
            k_ref, v_ref, gi_)

    @pl.when(g == NG - 1)
    def _global():
        slots = _gru_ff(upd_sc[...], slots_sc[...], *gru_args)

        @pl.when(j < 1)
        def _next():
            slots_sc[...] = slots
            q_sc[...] = _q_of(slots)

        @pl.when(j == 1)
        def _emit():
            out_ref[...] = slots.reshape(B, S, D)


@jax.jit
def kernel(x, slots_noise, mu, logsigma, Wq, bq, Wk, bk, Wv, bv,
           W_ih, W_hh, b_ih, b_hh, W1, b1, W2, b2,
           g_in, b_in, g_slots, b_slots, g_ff, b_ff):
    bf = jnp.bfloat16
    row = lambda a: a.reshape(1, -1)
    full = lambda s, n: pl.BlockSpec(s, lambda *_: (0,) * n)
    sigma = jnp.exp(logsigma)
    Wq_bf = Wq.astype(bf)
    Wk_eff = (Wk * g_in).astype(bf)
    Wv_eff = (Wv * g_in).astype(bf)
    bk_eff = row(bk + b_in @ Wk.T)
    bv_eff = row(bv + b_in @ Wv.T)

    k, v, upd0 = pl.pallas_call(
        _proj_kernel,
        grid=(NG, P),
        in_specs=[
            pl.BlockSpec((G, NH, D), lambda g, h: (g, h, 0)),
            pl.BlockSpec((G, S, D), lambda g, h: (g, 0, 0)),
            full((1, 1, D), 3), full((1, 1, D), 3),
            full((D, D), 2), full((1, D), 2),
            full((D, D), 2), full((1, D), 2),
            full((D, D), 2), full((1, D), 2),
            full((1, D), 2), full((1, D), 2),
        ],
        out_specs=[
            pl.BlockSpec((P, G, NH, D), lambda g, h: (0, g, 0, 0)),
            pl.BlockSpec((P, G, NH, D), lambda g, h: (0, g, 0, 0)),
            pl.BlockSpec((G, S, D), lambda g, h: (g, 0, 0)),
        ],
        out_shape=[
            jax.ShapeDtypeStruct((P, B, NH, D), bf),
            jax.ShapeDtypeStruct((P, B, NH, D), bf),
            jax.ShapeDtypeStruct((B, S, D), jnp.float32),
        ],
    )(x, slots_noise, mu, sigma,
      Wk_eff, bk_eff, Wv_eff, bv_eff, Wq_bf, row(bq),
      row(g_slots), row(b_slots))

    out = pl.pallas_call(
        _iter_kernel,
        grid=(ITERS - 1, NG),
        in_specs=[
            pl.BlockSpec((P, G, NH, D), lambda j, g: (0, g, 0, 0)),
            pl.BlockSpec((P, G, NH, D), lambda j, g: (0, g, 0, 0)),
            full((B, S, D), 3),
            full((B, S, D), 3),
            full((1, 1, D), 3), full((1, 1, D), 3),
            full((D, D), 2), full((1, D), 2),
            full((3 * D, D), 2), full((3 * D, D), 2),
            full((1, 3 * D), 2), full((1, 3 * D), 2),
            full((H, D), 2), full((1, H), 2),
            full((D, H), 2), full((1, D), 2),
            full((1, D), 2), full((1, D), 2),
            full((1, D), 2), full((1, D), 2),
        ],
        out_specs=full((B, S, D), 3),
        out_shape=jax.ShapeDtypeStruct((B, S, D), jnp.float32),
        scratch_shapes=[
            pltpu.VMEM((BS, D), jnp.float32),
            pltpu.VMEM((BS, D), jnp.float32),
            pltpu.VMEM((BS, D), jnp.float32),
        ],
    )(k, v, upd0, slots_noise, mu, sigma,
      Wq_bf, row(bq),
      W_ih.astype(bf), W_hh.astype(bf), row(b_ih), row(b_hh),
      W1.astype(bf), row(b1), W2.astype(bf), row(b2),
      row(g_slots), row(b_slots), row(g_ff), row(b_ff))
    return out


# speedup vs baseline: 1.0265x; 1.0037x over previous
import jax
import jax.numpy as jnp
from jax.experimental import pallas as pl
from jax.experimental.pallas import tpu as pltpu

B, N, D = 32, 1024, 768
S = 8
H = 1536
ITERS = 3
EPS = 1e-8
G = 4
NG = B // G
P = 2
NH = N // P
GS = G * S
BS = B * S
SCALE = D ** -0.5


def _dotT(a, w):
    return jax.lax.dot_general(a, w, (((1,), (1,)), ((), ())),
                               preferred_element_type=jnp.float32)


def _ln(x, g, b):
    m = jnp.mean(x, axis=-1, keepdims=True)
    v = jnp.mean((x - m) ** 2, axis=-1, keepdims=True)
    return (x - m) * jax.lax.rsqrt(v + 1e-5) * g + b


def _norm_rows(x):
    m = jnp.mean(x, axis=-1, keepdims=True)
    v = jnp.mean((x - m) ** 2, axis=-1, keepdims=True)
    return (x - m) * jax.lax.rsqrt(v + 1e-5)


def _soft_part(q_b, k_h, v_h):
    dots = jax.lax.dot_general(
        q_b, k_h, (((1,), (1,)), ((), ())),
        preferred_element_type=jnp.float32) * SCALE
    e = jnp.exp(dots - jnp.max(dots, axis=0, keepdims=True))
    p = e / jnp.sum(e, axis=0, keepdims=True)
    u = jnp.dot(p.astype(jnp.bfloat16), v_h,
                preferred_element_type=jnp.float32)
    return u, jnp.sum(p, axis=1, keepdims=True)


def _attend(q_b, kv_ref, gi_):
    us = [_soft_part(q_b, kv_ref[p_, gi_, :, :D], kv_ref[p_, gi_, :, D:])
          for p_ in range(P)]
    u = sum(x[0] for x in us)
    s = sum(x[1] for x in us)
    return u / (s + N * EPS)


def _proj_kernel(x_ref, noise_ref, mu_ref, sigma_ref,
                 Wk_ref, bk_ref, Wv_ref, bv_ref, Wq_ref, bq_ref,
                 g_s_ref, b_s_ref,
                 kv_ref, upd0_ref):
    bf = jnp.bfloat16
    h = pl.program_id(1)
    xh = _norm_rows(x_ref[...].reshape(G * NH, D)).astype(bf)
    kh = (_dotT(xh, Wk_ref[...]) + bk_ref[...]).astype(bf)
    vh = (_dotT(xh, Wv_ref[...]) + bv_ref[...]).astype(bf)
    kv_ref[pl.ds(h, 1), :, :, :D] = kh.reshape(1, G, NH, D)
    kv_ref[pl.ds(h, 1), :, :, D:] = vh.reshape(1, G, NH, D)

    @pl.when(h == P - 1)
    def _attn0():
        slots0 = mu_ref[0] + sigma_ref[0] * noise_ref[...].reshape(GS, D)
        q0 = (_dotT(_ln(slots0, g_s_ref[...], b_s_ref[...]).astype(bf),
                    Wq_ref[...]) + bq_ref[...]).astype(bf)
        for gi_ in range(G):
            upd0_ref[gi_] = _attend(q0[gi_ * S:(gi_ + 1) * S, :],
                                    kv_ref, gi_)


def _gru_ff(upd, slots_prev, Wih_ref, Whh_ref, bih_ref, bhh_ref,
            W1_ref, b1_ref, W2_ref, b2_ref, g_ff_ref, b_ff_ref):
    bf = jnp.bfloat16
    gi = _dotT(upd.astype(bf), Wih_ref[...]) + bih_ref[...]
    gh = _dotT(slots_prev.astype(bf), Whh_ref[...]) + bhh_ref[...]
    r = jax.nn.sigmoid(gi[:, :D] + gh[:, :D])
    z = jax.nn.sigmoid(gi[:, D:2 * D] + gh[:, D:2 * D])
    n_ = jnp.tanh(gi[:, 2 * D:] + r * gh[:, 2 * D:])
    slots = (1.0 - z) * n_ + z * slots_prev
    ffx = _ln(slots, g_ff_ref[...], b_ff_ref[...]).astype(bf)
    h1 = jax.nn.relu(_dotT(ffx, W1_ref[...]) + b1_ref[...]).astype(bf)
    return slots + _dotT(h1, W2_ref[...]) + b2_ref[...]


def _iter_kernel(kv_ref, upd0_ref, noise_ref, mu_ref, sigma_ref,
                 Wq_ref, bq_ref, Wih_ref, Whh_ref, bih_ref, bhh_ref,
                 W1_ref, b1_ref, W2_ref, b2_ref,
                 g_s_ref, b_s_ref, g_ff_ref, b_ff_ref,
                 out_ref, slots_sc, upd_sc, q_sc):
    bf = jnp.bfloat16
    j = pl.program_id(0)
    g = pl.program_id(1)
    gru_args = (Wih_ref, Whh_ref, bih_ref, bhh_ref,
                W1_ref, b1_ref, W2_ref, b2_ref, g_ff_ref, b_ff_ref)

    def _q_of(slots):
        return (_dotT(_ln(slots, g_s_ref[...], b_s_ref[...]).astype(bf),
                      Wq_ref[...]) + bq_ref[...])

    @pl.when(jnp.logical_and(j == 0, g == 0))
    def _init():
        slots0 = mu_ref[0] + sigma_ref[0] * noise_ref[...].reshape(BS, D)
        slots = _gru_ff(upd0_ref[...].reshape(BS, D), slots0, *gru_args)
        slots_sc[...] = slots
        q_sc[...] = _q_of(slots)

    for gi_ in range(G):
        upd_sc[pl.ds(g * GS + gi_ * S, S), :] = _attend(
            q_sc[pl.ds(g * GS + gi_ * S, S), :].astype(bf),
            kv_ref, gi_)

    @pl.when(g == NG - 1)
    def _global():
        slots = _gru_ff(upd_sc[...], slots_sc[...], *gru_args)

        @pl.when(j < 1)
        def _next():
            slots_sc[...] = slots
            q_sc[...] = _q_of(slots)

        @pl.when(j == 1)
        def _emit():
            out_ref[...] = slots.reshape(B, S, D)


@jax.jit
def kernel(x, slots_noise, mu, logsigma, Wq, bq, Wk, bk, Wv, bv,
           W_ih, W_hh, b_ih, b_hh, W1, b1, W2, b2,
           g_in, b_in, g_slots, b_slots, g_ff, b_ff):
    bf = jnp.bfloat16
    row = lambda a: a.reshape(1, -1)
    full = lambda s, n: pl.BlockSpec(s, lambda *_: (0,) * n)
    sigma = jnp.exp(logsigma)
    Wq_bf = Wq.astype(bf)
    Wk_eff = (Wk * g_in).astype(bf)
    Wv_eff = (Wv * g_in).astype(bf)
    bk_eff = row(bk + b_in @ Wk.T)
    bv_eff = row(bv + b_in @ Wv.T)

    kv, upd0 = pl.pallas_call(
        _proj_kernel,
        grid=(NG, P),
        in_specs=[
            pl.BlockSpec((G, NH, D), lambda g, h: (g, h, 0)),
            pl.BlockSpec((G, S, D), lambda g, h: (g, 0, 0)),
            full((1, 1, D), 3), full((1, 1, D), 3),
            full((D, D), 2), full((1, D), 2),
            full((D, D), 2), full((1, D), 2),
            full((D, D), 2), full((1, D), 2),
            full((1, D), 2), full((1, D), 2),
        ],
        out_specs=[
            pl.BlockSpec((P, G, NH, 2 * D), lambda g, h: (0, g, 0, 0)),
            pl.BlockSpec((G, S, D), lambda g, h: (g, 0, 0)),
        ],
        out_shape=[
            jax.ShapeDtypeStruct((P, B, NH, 2 * D), bf),
            jax.ShapeDtypeStruct((B, S, D), jnp.float32),
        ],
    )(x, slots_noise, mu, sigma,
      Wk_eff, bk_eff, Wv_eff, bv_eff, Wq_bf, row(bq),
      row(g_slots), row(b_slots))

    out = pl.pallas_call(
        _iter_kernel,
        grid=(ITERS - 1, NG),
        in_specs=[
            pl.BlockSpec((P, G, NH, 2 * D), lambda j, g: (0, g, 0, 0)),
            full((B, S, D), 3),
            full((B, S, D), 3),
            full((1, 1, D), 3), full((1, 1, D), 3),
            full((D, D), 2), full((1, D), 2),
            full((3 * D, D), 2), full((3 * D, D), 2),
            full((1, 3 * D), 2), full((1, 3 * D), 2),
            full((H, D), 2), full((1, H), 2),
            full((D, H), 2), full((1, D), 2),
            full((1, D), 2), full((1, D), 2),
            full((1, D), 2), full((1, D), 2),
        ],
        out_specs=full((B, S, D), 3),
        out_shape=jax.ShapeDtypeStruct((B, S, D), jnp.float32),
        scratch_shapes=[
            pltpu.VMEM((BS, D), jnp.float32),
            pltpu.VMEM((BS, D), jnp.float32),
            pltpu.VMEM((BS, D), jnp.float32),
        ],
    )(kv, upd0, slots_noise, mu, sigma,
      Wq_bf, row(bq),
      W_ih.astype(bf), W_hh.astype(bf), row(b_ih), row(b_hh),
      W1.astype(bf), row(b1), W2.astype(bf), row(b2),
      row(g_slots), row(b_slots), row(g_ff), row(b_ff))
    return out
